# baseline (speedup 1.0000x reference)
import functools

import jax
import jax.numpy as jnp
from jax import lax
from jax.experimental import pallas as pl
from jax.experimental.pallas import tpu as pltpu
from jax.experimental.pallas import tpu_sc as plsc

_NUM_ROWS = 1_000_000
_DIM = 10
_N = 16384

_CHUNK = 128
_NCHUNKS = _N // _CHUNK

_info = plsc.get_sparse_core_info()
_NC, _NS = _info.num_cores, _info.num_subcores
_NW = _NC * _NS
_CPW = _NCHUNKS // _NW

_mesh = plsc.VectorSubcoreMesh(core_axis_name="c", subcore_axis_name="s")


@functools.partial(
    pl.kernel,
    mesh=_mesh,
    compiler_params=pltpu.CompilerParams(use_tc_tiling_on_sc=False),
    out_type=jax.ShapeDtypeStruct((_NCHUNKS, _CHUNK, _DIM), jnp.float32),
    scratch_types=(
        [pltpu.VMEM((_CHUNK,), jnp.int32) for _ in range(_CPW)]
        + [pltpu.VMEM((_CHUNK, _DIM), jnp.float32) for _ in range(_CPW)]
        + [pltpu.SemaphoreType.DMA]
    ),
)
def _gather_kernel(table_hbm, idx_hbm, out_hbm, *scratch):
    idx_vs = scratch[:_CPW]
    row_vs = scratch[_CPW:2 * _CPW]
    sem = scratch[2 * _CPW]
    wid = lax.axis_index("s") * _NC + lax.axis_index("c")
    base = wid * _CPW
    for j in range(_CPW):
        pltpu.sync_copy(idx_hbm.at[base + j], idx_vs[j])
    copies = [
        pltpu.async_copy(table_hbm.at[idx_vs[j]], row_vs[j], sem)
        for j in range(_CPW)
    ]
    for cp in copies:
        cp.wait()
    for j in range(_CPW):
        pltpu.sync_copy(row_vs[j], out_hbm.at[base + j])


@jax.jit
def kernel(shape_param_human, rand_id):
    idx = rand_id.astype(jnp.int32).reshape(_NCHUNKS, _CHUNK)
    out = _gather_kernel(shape_param_human, idx)
    return out.reshape(_N, _DIM)
